# baseline (speedup 1.0000x reference)
import functools

import jax
import jax.numpy as jnp
from jax import lax
from jax.experimental import pallas as pl
from jax.experimental.pallas import tpu as pltpu
from jax.experimental.pallas import tpu_sc as plsc

B = 16384
EMB = 16
NC = 2
NS = 16
L = 16
NW = NC * NS
RPW = B // NW
CH = 128
NCH = RPW // CH

_mesh = plsc.VectorSubcoreMesh(core_axis_name="c", subcore_axis_name="s")


@functools.partial(
    pl.kernel,
    out_type=[
        jax.ShapeDtypeStruct((NW, L), jnp.float32),
        jax.ShapeDtypeStruct((B // CH, CH), jnp.float32),
    ],
    mesh=_mesh,
    compiler_params=pltpu.CompilerParams(use_tc_tiling_on_sc=False),
    scratch_types=[
        pltpu.VMEM((NCH, CH), jnp.int32),
        pltpu.VMEM((NCH, CH), jnp.int32),
        pltpu.VMEM((NCH, CH, EMB), jnp.float32),
        pltpu.VMEM((NCH, CH, EMB), jnp.float32),
        pltpu.VMEM((NCH, CH), jnp.float32),
        pltpu.VMEM((NCH, CH), jnp.float32),
        pltpu.VMEM((NCH, CH), jnp.float32),
        pltpu.VMEM((L,), jnp.float32),
        pltpu.SemaphoreType.DMA,
    ],
)
def _sc_gather_dot(uidx_hbm, iidx_hbm, uemb_hbm, iemb_hbm, ubias_hbm,
                   ibias_hbm, partials_hbm, bsum_hbm,
                   uidx_v, iidx_v, urows_v, irows_v, ub_v, ib_v, bs_v,
                   acc_v, sem):
    wid = lax.axis_index("s") * NC + lax.axis_index("c")
    rbase = wid * NCH

    pltpu.sync_copy(uidx_hbm.at[pl.ds(rbase, NCH)], uidx_v)
    pltpu.sync_copy(iidx_hbm.at[pl.ds(rbase, NCH)], iidx_v)

    copies = []
    for j in range(NCH):
        copies.append(pltpu.async_copy(uemb_hbm.at[uidx_v.at[j]], urows_v.at[j], sem))
        copies.append(pltpu.async_copy(iemb_hbm.at[iidx_v.at[j]], irows_v.at[j], sem))
        copies.append(pltpu.async_copy(ubias_hbm.at[uidx_v.at[j]], ub_v.at[j], sem))
        copies.append(pltpu.async_copy(ibias_hbm.at[iidx_v.at[j]], ib_v.at[j], sem))
    for c in copies:
        c.wait()

    acc = jnp.zeros((L,), jnp.float32)
    for j in range(NCH):
        def dot_body(i, a, j=j):
            return a + urows_v[j, i] * irows_v[j, i]
        acc = lax.fori_loop(0, CH, dot_body, acc)
    acc_v[...] = acc
    pltpu.sync_copy(acc_v, partials_hbm.at[wid])

    for j in range(NCH):
        def bias_body(cidx, _, j=j):
            sl = pl.ds(cidx * L, L)
            bs_v[j, sl] = ub_v[j, sl] + ib_v[j, sl]
            return 0
        lax.fori_loop(0, CH // L, bias_body, 0)
    pltpu.sync_copy(bs_v, bsum_hbm.at[pl.ds(rbase, NCH)])


def _tc_finalize(partials_ref, bsum_ref, out_ref):
    s = jnp.sum(partials_ref[...])
    out_ref[...] = jax.nn.sigmoid(s + bsum_ref[...])


def kernel(inputs, user_emb, user_bias, item_emb, item_bias):
    uidx = inputs[:, 0].reshape(B // CH, CH)
    iidx = inputs[:, 1].reshape(B // CH, CH)
    ub = user_bias.reshape(-1)
    ib = item_bias.reshape(-1)
    partials, bsum = _sc_gather_dot(uidx, iidx, user_emb, item_emb, ub, ib)
    out = pl.pallas_call(
        _tc_finalize,
        out_shape=jax.ShapeDtypeStruct((B // CH, CH), jnp.float32),
    )(partials, bsum)
    return out.reshape(B, 1)
